# baseline (speedup 1.0000x reference)
import functools

import jax
import jax.numpy as jnp
from jax import lax
from jax.experimental import pallas as pl
from jax.experimental.pallas import tpu as pltpu
from jax.experimental.pallas import tpu_sc as plsc

VECDIM = 32


@functools.cache
def _build_gather(B: int, D: int, chunk: int):
    info = plsc.get_sparse_core_info()
    nc, ns = info.num_cores, info.num_subcores
    nw = nc * ns
    assert B % nw == 0
    b_per_w = B // nw
    assert b_per_w % chunk == 0
    n_chunks = b_per_w // chunk
    mesh = plsc.VectorSubcoreMesh(core_axis_name="c", subcore_axis_name="s")

    @functools.partial(
        pl.kernel,
        mesh=mesh,
        out_type=jax.ShapeDtypeStruct((B, D), jnp.float32),
        scratch_types=[
            pltpu.VMEM((b_per_w,), jnp.int32),
            pltpu.VMEM((chunk, D), jnp.float32),
            pltpu.VMEM((chunk, D), jnp.float32),
            pltpu.SemaphoreType.DMA,
            pltpu.SemaphoreType.DMA,
            pltpu.SemaphoreType.DMA,
            pltpu.SemaphoreType.DMA,
        ],
        compiler_params=pltpu.CompilerParams(use_tc_tiling_on_sc=False),
    )
    def k(idx_hbm, table_hbm, out_hbm, idx_all, rows0, rows1,
          sg0, sg1, so0, so1):
        wid = lax.axis_index("s") * nc + lax.axis_index("c")
        base = wid * b_per_w
        rows = (rows0, rows1)
        sg = (sg0, sg1)
        so = (so0, so1)

        pltpu.sync_copy(idx_hbm.at[pl.ds(base, b_per_w)], idx_all)

        gather = [None, None]
        outcp = [None, None]
        gather[0] = pltpu.async_copy(
            table_hbm.at[idx_all.at[pl.ds(0, chunk)]], rows[0], sg[0])
        for c in range(n_chunks):
            b = c % 2
            gather[b].wait()
            outcp[b] = pltpu.async_copy(
                rows[b], out_hbm.at[pl.ds(base + c * chunk, chunk)], so[b])
            if c + 1 < n_chunks:
                if outcp[1 - b] is not None:
                    outcp[1 - b].wait()
                gather[1 - b] = pltpu.async_copy(
                    table_hbm.at[idx_all.at[pl.ds((c + 1) * chunk, chunk)]],
                    rows[1 - b], sg[1 - b])
        outcp[(n_chunks - 1) % 2].wait()
        if n_chunks > 1:
            outcp[n_chunks % 2].wait()

    return k


def kernel(x, table):
    bsz, hist = x.shape
    B = bsz * hist
    k = _build_gather(B, VECDIM, 1600)
    out = k(x.reshape(B), table)
    return out.reshape(bsz, hist, VECDIM)


# speedup vs baseline: 1.0052x; 1.0052x over previous
import functools

import jax
import jax.numpy as jnp
from jax import lax
from jax.experimental import pallas as pl
from jax.experimental.pallas import tpu as pltpu
from jax.experimental.pallas import tpu_sc as plsc

VECDIM = 32
NBUF = 8


@functools.cache
def _build_gather(bsz: int, hist: int, D: int, nbuf: int):
    info = plsc.get_sparse_core_info()
    nc, ns = info.num_cores, info.num_subcores
    nw = nc * ns
    assert bsz % nw == 0
    r_per_w = bsz // nw
    assert r_per_w % nbuf == 0
    n_groups = r_per_w // nbuf
    mesh = plsc.VectorSubcoreMesh(core_axis_name="c", subcore_axis_name="s")

    @functools.partial(
        pl.kernel,
        mesh=mesh,
        out_type=jax.ShapeDtypeStruct((bsz, hist, D), jnp.float32),
        scratch_types=[
            pltpu.VMEM((r_per_w, hist), jnp.int32),
            [pltpu.VMEM((hist, D), jnp.float32) for _ in range(nbuf)],
            [pltpu.SemaphoreType.DMA for _ in range(nbuf)],
            [pltpu.SemaphoreType.DMA for _ in range(nbuf)],
        ],
        compiler_params=pltpu.CompilerParams(use_tc_tiling_on_sc=False),
    )
    def k(x_hbm, table_hbm, out_hbm, idx2d, rows, sg, so):
        wid = lax.axis_index("s") * nc + lax.axis_index("c")
        row0 = wid * r_per_w

        pltpu.sync_copy(x_hbm.at[pl.ds(row0, r_per_w), :], idx2d)

        def start_gather(r, b):
            pltpu.async_copy(
                table_hbm.at[idx2d.at[r]], rows[b], sg[b])

        def drain(sem, b):
            pltpu.make_async_copy(
                table_hbm.at[pl.ds(0, hist)], rows[b], sem).wait()

        for b in range(nbuf):
            start_gather(b, b)

        def body(g, carry):
            for b in range(nbuf):
                drain(sg[b], b)
                pltpu.async_copy(rows[b], out_hbm.at[row0 + g * nbuf + b],
                                 so[b])

            @pl.when(g + 1 < n_groups)
            def _():
                for b in range(nbuf):
                    drain(so[b], b)
                    start_gather((g + 1) * nbuf + b, b)

            return carry

        lax.fori_loop(0, n_groups, body, 0)
        for b in range(nbuf):
            drain(so[b], b)

    return k


def kernel(x, table):
    bsz, hist = x.shape
    k = _build_gather(bsz, hist, VECDIM, NBUF)
    return k(x, table)
